# Initial kernel scaffold; baseline (speedup 1.0000x reference)
import jax
import jax.numpy as jnp
from jax.experimental import pallas as pl


def kernel(feature_data):
    raise NotImplementedError("write your pallas kernel here")


# speedup vs baseline: 14.8296x; 14.8296x over previous
import functools

import jax
import jax.numpy as jnp
from jax import lax
from jax.experimental import pallas as pl
from jax.experimental.pallas import tpu as pltpu
from jax.experimental.pallas import tpu_sc as plsc

_F32 = jnp.float32
_I32 = jnp.int32


def _vsqrt(x):
    xi = lax.bitcast_convert_type(x, _I32)
    yi = (xi >> 1) + jnp.int32(0x1FBD1DF5)
    y = lax.bitcast_convert_type(yi, _F32)
    for _ in range(4):
        y = 0.5 * (y + x / y)
    return y


def _splat_i32(v):
    return jnp.zeros((16,), _I32) + v


def _sc_body(fd_hbm, out_hbm, fdv, mt, d0, d1, d2, d3, pi, pj, hs, outv, sem):
    cid = lax.axis_index("c")
    sid = lax.axis_index("s")

    @pl.when((cid == 0) & (sid == 0))
    def _():
        pltpu.sync_copy(fd_hbm, fdv)
        lanes = lax.iota(_I32, 16)
        zeros = jnp.zeros((16,), _F32)

        def hsum_full(x):
            for s in (8, 4, 2, 1):
                hs[pl.ds(0, 16)] = x
                x = x + plsc.load_gather(hs, [lanes ^ s])
            return x

        def zero_row(r, _):
            mt[pl.ds(r * 16, 16)] = zeros
            return 0

        lax.fori_loop(0, 64, zero_row, 0)

        def m_problem(p, _):
            dirx = p < 4
            tb = p % 4
            t = 1 + tb // 2
            b = tb % 2
            rowmul = jnp.where(dirx, 128, 16)
            wmul = jnp.where(dirx, 16, 128)
            scale = jnp.where(dirx, _F32(0.125), _F32(1.0))
            base1 = b * 1024
            base2 = (t * 2 + b) * 1024

            def m_group(g, _):
                for k in range(16):
                    cell = g * 16 + k
                    i = cell // 8
                    j = cell % 8
                    offa = base1 + i * rowmul
                    offb = base2 + j * rowmul
                    acc = zeros
                    for w in range(8):
                        av = fdv[pl.ds(offa + w * wmul, 16)]
                        bv = fdv[pl.ds(offb + w * wmul, 16)]
                        dv = av - bv
                        acc = acc + dv * dv
                    hs[pl.ds(k * 16, 16)] = acc
                totals = zeros
                for ch in range(16):
                    totals = totals + plsc.load_gather(hs, [lanes * 16 + ch])
                mvec = _vsqrt(totals * scale) + _F32(1e-08)
                plsc.store_scatter(mt, [(g * 16 + lanes) * 16 + p], mvec)
                return 0

            lax.fori_loop(0, 4, m_group, 0)
            return 0

        lax.fori_loop(0, 8, m_problem, 0)

        d0[pl.ds(0, 16)] = mt[pl.ds(0, 16)]
        d1[pl.ds(0, 16)] = zeros - 1.0
        d2[pl.ds(0, 16)] = zeros - 1.0
        d3[pl.ds(0, 16)] = zeros + 1.0

        def i_edge(i, _):
            c = i * 8
            d0[pl.ds(c * 16, 16)] = mt[pl.ds(c * 16, 16)] + d0[pl.ds((c - 8) * 16, 16)]
            d1[pl.ds(c * 16, 16)] = zeros + (i - 1).astype(_F32)
            d2[pl.ds(c * 16, 16)] = zeros
            d3[pl.ds(c * 16, 16)] = d3[pl.ds((c - 8) * 16, 16)] + 1.0
            return 0

        lax.fori_loop(1, 8, i_edge, 0)

        def j_edge(j, _):
            d0[pl.ds(j * 16, 16)] = mt[pl.ds(j * 16, 16)] + d0[pl.ds((j - 1) * 16, 16)]
            d1[pl.ds(j * 16, 16)] = zeros
            d2[pl.ds(j * 16, 16)] = zeros + (j - 1).astype(_F32)
            d3[pl.ds(j * 16, 16)] = d3[pl.ds((j - 1) * 16, 16)] + 1.0
            return 0

        lax.fori_loop(1, 8, j_edge, 0)

        def dp_cell(q, _):
            i = 1 + q // 7
            j = 1 + q % 7
            c = i * 8 + j
            d0u = d0[pl.ds((c - 8) * 16, 16)]
            d0l = d0[pl.ds((c - 1) * 16, 16)]
            d0g = d0[pl.ds((c - 9) * 16, 16)]
            d3u = d3[pl.ds((c - 8) * 16, 16)]
            d3l = d3[pl.ds((c - 1) * 16, 16)]
            d3g = d3[pl.ds((c - 9) * 16, 16)]
            c1 = d0u / d3u
            c2 = d0l / d3l
            c3 = d0g / d3g
            b1 = (c1 <= c2) & (c1 <= c3)
            b2 = c2 <= c3
            mrow = mt[pl.ds(c * 16, 16)]
            d0[pl.ds(c * 16, 16)] = mrow + jnp.where(b1, d0u, jnp.where(b2, d0l, d0g))
            fi = i.astype(_F32)
            fj = j.astype(_F32)
            d1[pl.ds(c * 16, 16)] = jnp.where((~b1) & b2, fi, fi - 1.0)
            d2[pl.ds(c * 16, 16)] = jnp.where(b1, fj, fj - 1.0)
            d3[pl.ds(c * 16, 16)] = 1.0 + jnp.where(b1, d3u, jnp.where(b2, d3l, d3g))
            return 0

        lax.fori_loop(0, 49, dp_cell, 0)

        def bt_step(tt, carry):
            ii, jj = carry
            pi[pl.ds(tt * 16, 16)] = ii
            pj[pl.ds(tt * 16, 16)] = jj
            valid = ii >= 0
            ci = jnp.clip(ii, 0, 7)
            cj = jnp.clip(jj, 0, 7)
            idx = (ci * 8 + cj) * 16 + lanes
            n1 = plsc.load_gather(d1, [idx]).astype(_I32)
            n2 = plsc.load_gather(d2, [idx]).astype(_I32)
            return (jnp.where(valid, n1, ii), jnp.where(valid, n2, jj))

        seven = _splat_i32(7)
        lax.fori_loop(0, 16, bt_step, (seven, seven))

        loss = zeros
        for b in range(2):
            dists = []
            for t in (1, 2):
                xlane = (t - 1) * 2 + b
                ylane = 4 + (t - 1) * 2 + b
                yj_raw = plsc.load_gather(pi, [lanes * 16 + ylane])
                yb_raw = plsc.load_gather(pj, [lanes * 16 + ylane])
                ym = yj_raw >= 0
                jdx1 = b * 1024 + jnp.clip(yj_raw, 0, 7) * 16
                jdx2 = (t * 2 + b) * 1024 + jnp.clip(yb_raw, 0, 7) * 16

                def p_body(p, tv, xlane=xlane, ym=ym, jdx1=jdx1, jdx2=jdx2):
                    xr = plsc.load_gather(pi, [_splat_i32(p * 16 + xlane)])
                    ar = plsc.load_gather(pj, [_splat_i32(p * 16 + xlane)])
                    xm = xr >= 0
                    idx1 = jdx1 + jnp.clip(xr, 0, 7) * 128
                    idx2 = jdx2 + jnp.clip(ar, 0, 7) * 128
                    accq = zeros
                    for ch in range(16):
                        g1 = plsc.load_gather(fdv, [idx1 + ch])
                        g2 = plsc.load_gather(fdv, [idx2 + ch])
                        dv = g1 - g2
                        accq = accq + dv * dv
                    ddq = _vsqrt(accq)
                    return tv + jnp.where(xm & ym, ddq, zeros)

                tvec = lax.fori_loop(0, 15, p_body, zeros)
                total = hsum_full(tvec)
                xr_all = plsc.load_gather(pi, [lanes * 16 + xlane])
                cx = hsum_full(jnp.where(xr_all >= 0, _F32(1.0), _F32(0.0)))
                cy = hsum_full(jnp.where(ym, _F32(1.0), _F32(0.0)))
                dists.append(total / (cx * cy))
            loss = loss + jnp.maximum(dists[0] - dists[1] + _F32(0.1), zeros)

        outv[...] = loss
        pltpu.sync_copy(outv, out_hbm)


@jax.jit
def _run(fd_flat):
    mesh = plsc.VectorSubcoreMesh(core_axis_name="c", subcore_axis_name="s")
    k = functools.partial(
        pl.kernel,
        mesh=mesh,
        out_type=jax.ShapeDtypeStruct((16,), _F32),
        compiler_params=pltpu.CompilerParams(needs_layout_passes=False),
        scratch_types=[
            pltpu.VMEM((6144,), _F32),
            pltpu.VMEM((1024,), _F32),
            pltpu.VMEM((1024,), _F32),
            pltpu.VMEM((1024,), _F32),
            pltpu.VMEM((1024,), _F32),
            pltpu.VMEM((1024,), _F32),
            pltpu.VMEM((256,), _I32),
            pltpu.VMEM((256,), _I32),
            pltpu.VMEM((256,), _F32),
            pltpu.VMEM((16,), _F32),
            pltpu.SemaphoreType.DMA,
        ],
    )(_sc_body)
    return k(fd_flat)


def kernel(feature_data):
    fd_flat = jnp.asarray(feature_data, dtype=_F32).reshape(6144)
    out = _run(fd_flat)
    return out[:1]


# speedup vs baseline: 24.4434x; 1.6483x over previous
import functools

import jax
import jax.numpy as jnp
from jax import lax
from jax.experimental import pallas as pl
from jax.experimental.pallas import tpu as pltpu
from jax.experimental.pallas import tpu_sc as plsc

_F32 = jnp.float32
_I32 = jnp.int32


def _vsqrt(x):
    xi = lax.bitcast_convert_type(x, _I32)
    yi = (xi >> 1) + jnp.int32(0x1FBD1DF5)
    y = lax.bitcast_convert_type(yi, _F32)
    for _ in range(4):
        y = 0.5 * (y + x / y)
    return y


def _splat_i32(v):
    return jnp.zeros((16,), _I32) + v


def _sc_body(fd_hbm, out_hbm, fdv, mt, d0, d1, d2, d3, pi, pj, hs, mcol,
             outv, shm, shp, shd, sem):
    cid = lax.axis_index("c")
    sid = lax.axis_index("s")
    lanes = lax.iota(_I32, 16)
    zeros = jnp.zeros((16,), _F32)

    def hsum_full(x):
        for s in (8, 4, 2, 1):
            hs[pl.ds(0, 16)] = x
            x = x + plsc.load_gather(hs, [lanes ^ s])
        return x

    @pl.when((cid == 0) & (sid < 8))
    def _():
        pltpu.sync_copy(fd_hbm, fdv)
        p = sid
        dirx = p < 4
        tb = p % 4
        t = 1 + tb // 2
        b = tb % 2
        rowmul = jnp.where(dirx, 128, 16)
        wmul = jnp.where(dirx, 16, 128)
        scale = jnp.where(dirx, _F32(0.125), _F32(1.0))
        base1 = b * 1024
        base2 = (t * 2 + b) * 1024

        def m_group(g, _):
            for k in range(16):
                cell = g * 16 + k
                i = cell // 8
                j = cell % 8
                offa = base1 + i * rowmul
                offb = base2 + j * rowmul
                acc = zeros
                for w in range(8):
                    av = fdv[pl.ds(offa + w * wmul, 16)]
                    bv = fdv[pl.ds(offb + w * wmul, 16)]
                    dv = av - bv
                    acc = acc + dv * dv
                hs[pl.ds(k * 16, 16)] = acc
            totals = zeros
            for ch in range(16):
                totals = totals + plsc.load_gather(hs, [lanes * 16 + ch])
            mvec = _vsqrt(totals * scale) + _F32(1e-08)
            mcol[pl.ds(g * 16, 16)] = mvec
            return 0

        lax.fori_loop(0, 4, m_group, 0)
        pltpu.sync_copy(mcol.at[pl.ds(0, 64)], shm.at[pl.ds(p * 64, 64)])

    plsc.subcore_barrier()

    @pl.when((cid == 0) & (sid == 0))
    def _():
        def zero_hi(r, _):
            mcol[pl.ds(512 + r * 16, 16)] = zeros
            return 0

        lax.fori_loop(0, 32, zero_hi, 0)
        pltpu.sync_copy(shm, mcol.at[pl.ds(0, 512)])

        def mt_row(c, _):
            mt[pl.ds(c * 16, 16)] = plsc.load_gather(mcol, [lanes * 64 + c])
            return 0

        lax.fori_loop(0, 64, mt_row, 0)

        d0[pl.ds(0, 16)] = mt[pl.ds(0, 16)]
        d1[pl.ds(0, 16)] = zeros - 1.0
        d2[pl.ds(0, 16)] = zeros - 1.0
        d3[pl.ds(0, 16)] = zeros + 1.0

        def i_edge(i, _):
            c = i * 8
            d0[pl.ds(c * 16, 16)] = mt[pl.ds(c * 16, 16)] + d0[pl.ds((c - 8) * 16, 16)]
            d1[pl.ds(c * 16, 16)] = zeros + (i - 1).astype(_F32)
            d2[pl.ds(c * 16, 16)] = zeros
            d3[pl.ds(c * 16, 16)] = d3[pl.ds((c - 8) * 16, 16)] + 1.0
            return 0

        lax.fori_loop(1, 8, i_edge, 0)

        def j_edge(j, _):
            d0[pl.ds(j * 16, 16)] = mt[pl.ds(j * 16, 16)] + d0[pl.ds((j - 1) * 16, 16)]
            d1[pl.ds(j * 16, 16)] = zeros
            d2[pl.ds(j * 16, 16)] = zeros + (j - 1).astype(_F32)
            d3[pl.ds(j * 16, 16)] = d3[pl.ds((j - 1) * 16, 16)] + 1.0
            return 0

        lax.fori_loop(1, 8, j_edge, 0)

        def dp_cell(q, _):
            i = 1 + q // 7
            j = 1 + q % 7
            c = i * 8 + j
            d0u = d0[pl.ds((c - 8) * 16, 16)]
            d0l = d0[pl.ds((c - 1) * 16, 16)]
            d0g = d0[pl.ds((c - 9) * 16, 16)]
            d3u = d3[pl.ds((c - 8) * 16, 16)]
            d3l = d3[pl.ds((c - 1) * 16, 16)]
            d3g = d3[pl.ds((c - 9) * 16, 16)]
            c1 = d0u / d3u
            c2 = d0l / d3l
            c3 = d0g / d3g
            b1 = (c1 <= c2) & (c1 <= c3)
            b2 = c2 <= c3
            mrow = mt[pl.ds(c * 16, 16)]
            d0[pl.ds(c * 16, 16)] = mrow + jnp.where(b1, d0u, jnp.where(b2, d0l, d0g))
            fi = i.astype(_F32)
            fj = j.astype(_F32)
            d1[pl.ds(c * 16, 16)] = jnp.where((~b1) & b2, fi, fi - 1.0)
            d2[pl.ds(c * 16, 16)] = jnp.where(b1, fj, fj - 1.0)
            d3[pl.ds(c * 16, 16)] = 1.0 + jnp.where(b1, d3u, jnp.where(b2, d3l, d3g))
            return 0

        lax.fori_loop(0, 49, dp_cell, 0)

        def bt_step(tt, carry):
            ii, jj = carry
            pi[pl.ds(tt * 16, 16)] = ii
            pj[pl.ds(tt * 16, 16)] = jj
            valid = ii >= 0
            ci = jnp.clip(ii, 0, 7)
            cj = jnp.clip(jj, 0, 7)
            idx = (ci * 8 + cj) * 16 + lanes
            n1 = plsc.load_gather(d1, [idx]).astype(_I32)
            n2 = plsc.load_gather(d2, [idx]).astype(_I32)
            return (jnp.where(valid, n1, ii), jnp.where(valid, n2, jj))

        seven = _splat_i32(7)
        lax.fori_loop(0, 16, bt_step, (seven, seven))
        pltpu.sync_copy(pi, shp.at[pl.ds(0, 256)])
        pltpu.sync_copy(pj, shp.at[pl.ds(256, 256)])

    plsc.subcore_barrier()

    @pl.when((cid == 0) & (sid < 4))
    def _():
        pltpu.sync_copy(shp.at[pl.ds(0, 256)], pi)
        pltpu.sync_copy(shp.at[pl.ds(256, 256)], pj)
        b = sid & 1
        t = 1 + (sid >> 1)
        xlane = (t - 1) * 2 + b
        ylane = 4 + xlane
        yj_raw = plsc.load_gather(pi, [lanes * 16 + ylane])
        yb_raw = plsc.load_gather(pj, [lanes * 16 + ylane])
        ym = yj_raw >= 0
        jdx1 = b * 1024 + jnp.clip(yj_raw, 0, 7) * 16
        jdx2 = (t * 2 + b) * 1024 + jnp.clip(yb_raw, 0, 7) * 16

        def p_body(p, tv):
            xr = plsc.load_gather(pi, [_splat_i32(p * 16 + xlane)])
            ar = plsc.load_gather(pj, [_splat_i32(p * 16 + xlane)])
            xm = xr >= 0
            idx1 = jdx1 + jnp.clip(xr, 0, 7) * 128
            idx2 = jdx2 + jnp.clip(ar, 0, 7) * 128
            accq = zeros
            for ch in range(16):
                g1 = plsc.load_gather(fdv, [idx1 + ch])
                g2 = plsc.load_gather(fdv, [idx2 + ch])
                dv = g1 - g2
                accq = accq + dv * dv
            ddq = _vsqrt(accq)
            return tv + jnp.where(xm & ym, ddq, zeros)

        tvec = lax.fori_loop(0, 15, p_body, zeros)
        total = hsum_full(tvec)
        xr_all = plsc.load_gather(pi, [lanes * 16 + xlane])
        cx = hsum_full(jnp.where(xr_all >= 0, _F32(1.0), _F32(0.0)))
        cy = hsum_full(jnp.where(ym, _F32(1.0), _F32(0.0)))
        outv[...] = total / (cx * cy)
        pltpu.sync_copy(outv, shd.at[pl.ds(sid * 16, 16)])

    plsc.subcore_barrier()

    @pl.when((cid == 0) & (sid == 0))
    def _():
        pltpu.sync_copy(shd, hs.at[pl.ds(0, 64)])
        dp0 = hs[pl.ds(0, 16)]
        dp1 = hs[pl.ds(16, 16)]
        dn0 = hs[pl.ds(32, 16)]
        dn1 = hs[pl.ds(48, 16)]
        loss = (jnp.maximum(dp0 - dn0 + _F32(0.1), zeros)
                + jnp.maximum(dp1 - dn1 + _F32(0.1), zeros))
        outv[...] = loss
        pltpu.sync_copy(outv, out_hbm)


@jax.jit
def _run(fd_flat):
    mesh = plsc.VectorSubcoreMesh(core_axis_name="c", subcore_axis_name="s")
    k = functools.partial(
        pl.kernel,
        mesh=mesh,
        out_type=jax.ShapeDtypeStruct((16,), _F32),
        compiler_params=pltpu.CompilerParams(needs_layout_passes=False),
        scratch_types=[
            pltpu.VMEM((6144,), _F32),
            pltpu.VMEM((1024,), _F32),
            pltpu.VMEM((1024,), _F32),
            pltpu.VMEM((1024,), _F32),
            pltpu.VMEM((1024,), _F32),
            pltpu.VMEM((1024,), _F32),
            pltpu.VMEM((256,), _I32),
            pltpu.VMEM((256,), _I32),
            pltpu.VMEM((256,), _F32),
            pltpu.VMEM((1024,), _F32),
            pltpu.VMEM((16,), _F32),
            pltpu.VMEM_SHARED((512,), _F32),
            pltpu.VMEM_SHARED((512,), _I32),
            pltpu.VMEM_SHARED((64,), _F32),
            pltpu.SemaphoreType.DMA,
        ],
    )(_sc_body)
    return k(fd_flat)


def kernel(feature_data):
    fd_flat = jnp.asarray(feature_data, dtype=_F32).reshape(6144)
    out = _run(fd_flat)
    return out[:1]
